# baseline (speedup 1.0000x reference)
import functools

import jax
import jax.numpy as jnp
from jax import lax
from jax.experimental import pallas as pl
from jax.experimental.pallas import tpu as pltpu
from jax.experimental.pallas import tpu_sc as plsc

EMBED_DIM = 128
VOCAB = 11
BATCH = 16384

_NC = 2
_NS = 16
_NW = _NC * _NS
_B_PER_W = BATCH // _NW
_LANES = 16
_NCHUNK = 8
_CROWS = _B_PER_W // _NCHUNK


def _rsqrt_newton(v):
    i = lax.bitcast_convert_type(v, jnp.int32)
    y = lax.bitcast_convert_type(0x5F3759DF - (i >> 1), jnp.float32)
    for _ in range(3):
        y = y * (1.5 - 0.5 * v * y * y)
    return y


def _lane_sum(x):
    for w in (64, 32, 16, 8, 4, 2, 1):
        x = x[:w] + x[w:]
    return x


_sc_mesh = plsc.VectorSubcoreMesh(core_axis_name="c", subcore_axis_name="s")


@functools.partial(
    pl.kernel,
    mesh=_sc_mesh,
    out_type=jax.ShapeDtypeStruct((BATCH, EMBED_DIM), jnp.float32),
    scratch_types=[
        pltpu.VMEM((VOCAB, EMBED_DIM), jnp.float32),
        pltpu.VMEM((VOCAB, EMBED_DIM), jnp.float32),
        pltpu.VMEM((EMBED_DIM,), jnp.float32),
        pltpu.VMEM((EMBED_DIM,), jnp.float32),
        pltpu.VMEM((_B_PER_W,), jnp.int32),
        pltpu.VMEM((_B_PER_W, EMBED_DIM), jnp.float32),
        pltpu.SemaphoreType.DMA,
    ],
)
def _sc_all(table_hbm, gamma_hbm, beta_hbm, lab_hbm, out_hbm,
            traw_v, tnorm_v, gamma_v, beta_v, lab_v, rows_v, sem):
    wid = lax.axis_index("s") * _NC + lax.axis_index("c")
    base = wid * _B_PER_W
    pltpu.sync_copy(table_hbm, traw_v)
    pltpu.sync_copy(gamma_hbm, gamma_v)
    pltpu.sync_copy(beta_hbm, beta_v)
    pltpu.sync_copy(lab_hbm.at[pl.ds(base, _B_PER_W)], lab_v)

    gamma = gamma_v[:]
    beta = beta_v[:]
    for r in range(VOCAB):
        row = traw_v[r, :]
        mean = _lane_sum(row) * (1.0 / EMBED_DIM)
        d = row - mean
        var = _lane_sum(d * d) * (1.0 / EMBED_DIM)
        rs = _rsqrt_newton(var + 1e-5)
        tnorm_v[r, :] = d * rs * gamma + beta

    copies = []
    for ch in range(_NCHUNK):
        def body(c, _, ch=ch):
            g = ch * _CROWS + c * _LANES
            lv = lab_v[pl.ds(g, _LANES)]
            for l in range(_LANES):
                rows_v[g + l, :] = tnorm_v[lv[l], :]
            return _

        lax.fori_loop(0, _CROWS // _LANES, body, None)
        copies.append(pltpu.async_copy(
            rows_v.at[pl.ds(ch * _CROWS, _CROWS)],
            out_hbm.at[pl.ds(base + ch * _CROWS, _CROWS)],
            sem,
        ))
    for cp in copies:
        cp.wait()


def kernel(labels, table, gamma, beta):
    out = _sc_all(table, gamma, beta, labels.astype(jnp.int32))
    return out.reshape(BATCH, 1, EMBED_DIM)


# speedup vs baseline: 1.7764x; 1.7764x over previous
import jax
import jax.numpy as jnp
from jax import lax
from jax.experimental import pallas as pl

EMBED_DIM = 128
VOCAB = 11
BATCH = 16384
_VPAD = 16
_GRID = 16
_BLK = BATCH // _GRID
_JCOL = _BLK // 128


def _tc_body(lt_ref, tpad_ref, gamma_ref, beta_ref, out_ref):
    x = tpad_ref[...]
    mean = jnp.mean(x, axis=1, keepdims=True)
    d = x - mean
    var = jnp.mean(d * d, axis=1, keepdims=True)
    norm = d * lax.rsqrt(var + 1e-5) * gamma_ref[...] + beta_ref[...]

    vio = lax.broadcasted_iota(jnp.int32, (1, _VPAD), 1)
    for j in range(_JCOL):
        col = lt_ref[0, :, j:j + 1]
        oh = jnp.where(col == vio, 1.0, 0.0)
        out_ref[pl.ds(128 * j, 128), :] = jnp.dot(
            oh, norm, preferred_element_type=jnp.float32,
            precision=lax.Precision.HIGHEST)


_tc_onehot = pl.pallas_call(
    _tc_body,
    grid=(_GRID,),
    in_specs=[
        pl.BlockSpec((1, 128, _JCOL), lambda g: (g, 0, 0)),
        pl.BlockSpec((_VPAD, EMBED_DIM), lambda g: (0, 0)),
        pl.BlockSpec((1, EMBED_DIM), lambda g: (0, 0)),
        pl.BlockSpec((1, EMBED_DIM), lambda g: (0, 0)),
    ],
    out_specs=pl.BlockSpec((_BLK, EMBED_DIM), lambda g: (g, 0)),
    out_shape=jax.ShapeDtypeStruct((BATCH, EMBED_DIM), jnp.float32),
)


def kernel(labels, table, gamma, beta):
    tpad = jnp.zeros((_VPAD, EMBED_DIM), jnp.float32).at[:VOCAB].set(table)
    lt = labels.astype(jnp.int32).reshape(_GRID, _JCOL, 128).transpose(0, 2, 1)
    out = _tc_onehot(
        lt, tpad, gamma.reshape(1, EMBED_DIM), beta.reshape(1, EMBED_DIM)
    )
    return out.reshape(BATCH, 1, EMBED_DIM)
